# baseline (speedup 1.0000x reference)
import numpy as np
import jax
import jax.numpy as jnp
from jax import lax
from jax.experimental import pallas as pl
from jax.experimental.pallas import tpu as pltpu

_VOCAB = 1024
_D = 256
_LEVELS = 2
_KL_WEIGHT = 0.001
_ROWS = 8 * 1024
_BN = 1024
_LOG2E = np.float32(np.log2(np.e))


def _hqvae_block(zt_ref, zb_ref, ct_ref, cbm_ref, out_ref, kl_ref, com_ref,
                 cb2_ref, c2row_ref, caug_ref):
    @pl.when(pl.program_id(0) == 0)
    def _prep():
        for t, cref in enumerate((ct_ref, cbm_ref)):
            cb = cref[...]
            cbsq = cb * cb
            cb2_ref[t] = (cb * (2.0 * _LOG2E)).astype(jnp.bfloat16)
            c2row_ref[t:t + 1, :] = lax.dot_general(
                jnp.full((1, _D), _LOG2E, jnp.float32), cbsq,
                (((1,), (1,)), ((), ())),
                preferred_element_type=jnp.float32)
            caug_ref[t, :, :_D] = cb.astype(jnp.bfloat16)
            caug_ref[t, :, _D:_D + 1] = jnp.ones((_VOCAB, 1), jnp.bfloat16)
            caug_ref[t, :, _D + 1:_D + 2] = jnp.sum(
                cbsq, axis=1, keepdims=True).astype(jnp.bfloat16)

    log_k = np.float32(np.log(float(_VOCAB)))

    z = [zt_ref[...], zb_ref[...]]
    r = [z[0], z[1]]
    kl = [jnp.float32(0.0), jnp.float32(0.0)]
    for _ in range(_LEVELS):
        e = [None, None]
        u = [None, None]
        for t in range(2):
            e[t] = jnp.exp2(lax.dot_general(
                r[t].astype(jnp.bfloat16), cb2_ref[t],
                (((1,), (1,)), ((), ())),
                preferred_element_type=jnp.float32)
                - c2row_ref[t:t + 1, :]).astype(jnp.bfloat16)
        for t in range(2):
            u[t] = lax.dot_general(
                e[t], caug_ref[t], (((1,), (0,)), ((), ())),
                preferred_element_type=jnp.float32)
        for t in range(2):
            zsum = u[t][:, _D:_D + 1]
            ec2 = u[t][:, _D + 1:_D + 2]
            inv = 1.0 / zsum
            q = u[t][:, :_D] * inv
            kl[t] = (kl[t]
                     + 2.0 * jnp.sum(r[t] * q)
                     - jnp.sum(ec2 * inv + jnp.log(zsum))
                     + np.float32(_BN) * log_k)
            r[t] = r[t] - q

    for t in range(2):
        out_ref[:, t * _D:(t + 1) * _D] = z[t] - r[t]
    kl_ref[...] = (kl[0] + kl[1]).reshape(1, 1, 1)
    com_ref[...] = (jnp.sum(r[0] * r[0])
                    + jnp.sum(r[1] * r[1])).reshape(1, 1, 1)


def kernel(top_latent, bottom_latent, top_codebook, bottom_codebook):
    nb = _ROWS // _BN

    quant, kl_parts, com_parts = pl.pallas_call(
        _hqvae_block,
        grid=(nb,),
        in_specs=[
            pl.BlockSpec((_BN, _D), lambda i: (i, 0)),
            pl.BlockSpec((_BN, _D), lambda i: (i, 0)),
            pl.BlockSpec((_VOCAB, _D), lambda i: (0, 0)),
            pl.BlockSpec((_VOCAB, _D), lambda i: (0, 0)),
        ],
        out_specs=(
            pl.BlockSpec((_BN, 2 * _D), lambda i: (i, 0)),
            pl.BlockSpec((1, 1, 1), lambda i: (i, 0, 0)),
            pl.BlockSpec((1, 1, 1), lambda i: (i, 0, 0)),
        ),
        out_shape=(
            jax.ShapeDtypeStruct((_ROWS, 2 * _D), jnp.float32),
            jax.ShapeDtypeStruct((nb, 1, 1), jnp.float32),
            jax.ShapeDtypeStruct((nb, 1, 1), jnp.float32),
        ),
        scratch_shapes=[
            pltpu.VMEM((2, _VOCAB, _D), jnp.bfloat16),
            pltpu.VMEM((2, _VOCAB), jnp.float32),
            pltpu.VMEM((2, _VOCAB, _D + 2), jnp.bfloat16),
        ],
    )(top_latent.reshape(_ROWS, _D), bottom_latent.reshape(_ROWS, _D),
      top_codebook, bottom_codebook)

    loss = (jnp.sum(com_parts) / np.float32(_ROWS * _D)
            + np.float32(_KL_WEIGHT) * jnp.sum(kl_parts) / np.float32(_ROWS))
    return (loss, quant.reshape(8, 1024, 2 * _D))


# speedup vs baseline: 1.0257x; 1.0257x over previous
import numpy as np
import jax
import jax.numpy as jnp
from jax import lax
from jax.experimental import pallas as pl
from jax.experimental.pallas import tpu as pltpu

_VOCAB = 1024
_D = 256
_LEVELS = 2
_KL_WEIGHT = 0.001
_ROWS = 8 * 1024
_BN = 1024
_LOG2E = np.float32(np.log2(np.e))


def _hqvae_block(zt_ref, zb_ref, ct_ref, cbm_ref, out_ref, kl_ref, com_ref,
                 cb2_ref, c2row_ref, caug_ref):
    @pl.when(pl.program_id(0) == 0)
    def _prep():
        for t, cref in enumerate((ct_ref, cbm_ref)):
            cb = cref[...]
            cbsq = cb * cb
            cb2_ref[t] = (cb * (2.0 * _LOG2E)).astype(jnp.bfloat16)
            c2row_ref[t:t + 1, :] = lax.dot_general(
                jnp.full((1, _D), _LOG2E, jnp.float32), cbsq,
                (((1,), (1,)), ((), ())),
                preferred_element_type=jnp.float32)
            caug_ref[t, :, :_D] = cb.astype(jnp.bfloat16)
            caug_ref[t, :, _D:_D + 1] = jnp.ones((_VOCAB, 1), jnp.bfloat16)
            caug_ref[t, :, _D + 1:_D + 2] = jnp.sum(
                cbsq, axis=1, keepdims=True).astype(jnp.bfloat16)

    log_k = np.float32(np.log(float(_VOCAB)))

    z = [zt_ref[...], zb_ref[...]]
    r = [z[0], z[1]]
    kl = [jnp.float32(0.0), jnp.float32(0.0)]
    for _ in range(_LEVELS):
        e = [None, None]
        u = [None, None]
        for t in range(2):
            e[t] = jnp.exp2(lax.dot_general(
                r[t].astype(jnp.bfloat16), cb2_ref[t],
                (((1,), (1,)), ((), ())),
                preferred_element_type=jnp.float32)
                - c2row_ref[t:t + 1, :]).astype(jnp.bfloat16)
        for t in range(2):
            u[t] = lax.dot_general(
                e[t], caug_ref[t], (((1,), (0,)), ((), ())),
                preferred_element_type=jnp.float32)
        for t in range(2):
            zsum = u[t][:, _D:_D + 1]
            ec2 = u[t][:, _D + 1:_D + 2]
            inv = 1.0 / zsum
            q = u[t][:, :_D] * inv
            kl[t] = (kl[t]
                     + 2.0 * jnp.sum(r[t] * q)
                     - jnp.sum(ec2 * inv)
                     - jnp.sum(jnp.log(zsum))
                     + np.float32(_BN) * log_k)
            r[t] = r[t] - q

    for t in range(2):
        out_ref[:, t * _D:(t + 1) * _D] = z[t] - r[t]
    kl_ref[...] = (kl[0] + kl[1]).reshape(1, 1, 1)
    com_ref[...] = (jnp.sum(r[0] * r[0])
                    + jnp.sum(r[1] * r[1])).reshape(1, 1, 1)


def kernel(top_latent, bottom_latent, top_codebook, bottom_codebook):
    nb = _ROWS // _BN

    quant, kl_parts, com_parts = pl.pallas_call(
        _hqvae_block,
        grid=(nb,),
        in_specs=[
            pl.BlockSpec((_BN, _D), lambda i: (i, 0)),
            pl.BlockSpec((_BN, _D), lambda i: (i, 0)),
            pl.BlockSpec((_VOCAB, _D), lambda i: (0, 0)),
            pl.BlockSpec((_VOCAB, _D), lambda i: (0, 0)),
        ],
        out_specs=(
            pl.BlockSpec((_BN, 2 * _D), lambda i: (i, 0)),
            pl.BlockSpec((1, 1, 1), lambda i: (i, 0, 0)),
            pl.BlockSpec((1, 1, 1), lambda i: (i, 0, 0)),
        ),
        out_shape=(
            jax.ShapeDtypeStruct((_ROWS, 2 * _D), jnp.float32),
            jax.ShapeDtypeStruct((nb, 1, 1), jnp.float32),
            jax.ShapeDtypeStruct((nb, 1, 1), jnp.float32),
        ),
        scratch_shapes=[
            pltpu.VMEM((2, _VOCAB, _D), jnp.bfloat16),
            pltpu.VMEM((2, _VOCAB), jnp.float32),
            pltpu.VMEM((2, _VOCAB, _D + 2), jnp.bfloat16),
        ],
    )(top_latent.reshape(_ROWS, _D), bottom_latent.reshape(_ROWS, _D),
      top_codebook, bottom_codebook)

    loss = (jnp.sum(com_parts) / np.float32(_ROWS * _D)
            + np.float32(_KL_WEIGHT) * jnp.sum(kl_parts) / np.float32(_ROWS))
    return (loss, quant.reshape(8, 1024, 2 * _D))


# speedup vs baseline: 1.0533x; 1.0269x over previous
import numpy as np
import jax
import jax.numpy as jnp
from jax import lax
from jax.experimental import pallas as pl
from jax.experimental.pallas import tpu as pltpu

_VOCAB = 1024
_D = 256
_LEVELS = 2
_KL_WEIGHT = 0.001
_ROWS = 8 * 1024
_BN = 1024
_LOG2E = np.float32(np.log2(np.e))


def _hqvae_block(zt_ref, zb_ref, ct_ref, cbm_ref, out_ref, kl_ref, com_ref,
                 cb2_ref, c2row_ref, caug_ref, c2n_ref):
    @pl.when(pl.program_id(0) == 0)
    def _prep():
        for t, cref in enumerate((ct_ref, cbm_ref)):
            cb = cref[...]
            cbsq = cb * cb
            cb2_ref[t] = (cb * (2.0 * _LOG2E)).astype(jnp.bfloat16)
            c2row_ref[t:t + 1, :] = lax.dot_general(
                jnp.full((1, _D), _LOG2E, jnp.float32), cbsq,
                (((1,), (1,)), ((), ())),
                preferred_element_type=jnp.float32)
            caug_ref[t] = cb.astype(jnp.bfloat16)
            c2n_ref[t:t + 1, :] = lax.dot_general(
                jnp.ones((1, _D), jnp.float32), cbsq,
                (((1,), (1,)), ((), ())),
                preferred_element_type=jnp.float32)

    log_k = np.float32(np.log(float(_VOCAB)))

    z = [zt_ref[...], zb_ref[...]]
    r = [z[0], z[1]]
    kl = [jnp.float32(0.0), jnp.float32(0.0)]
    for _ in range(_LEVELS):
        e = [None, None]
        u = [None, None]
        for t in range(2):
            e[t] = jnp.exp2(lax.dot_general(
                r[t].astype(jnp.bfloat16), cb2_ref[t],
                (((1,), (1,)), ((), ())),
                preferred_element_type=jnp.float32)
                - c2row_ref[t:t + 1, :]).astype(jnp.bfloat16)
        for t in range(2):
            u[t] = lax.dot_general(
                e[t], caug_ref[t], (((1,), (0,)), ((), ())),
                preferred_element_type=jnp.float32)
        for t in range(2):
            ef = e[t].astype(jnp.float32)
            zsum = jnp.sum(ef, axis=1, keepdims=True)
            ec2 = jnp.sum(ef * c2n_ref[t:t + 1, :], axis=1, keepdims=True)
            inv = 1.0 / zsum
            q = u[t] * inv
            kl[t] = (kl[t]
                     + 2.0 * jnp.sum(r[t] * q)
                     - jnp.sum(ec2 * inv)
                     - jnp.sum(jnp.log(zsum))
                     + np.float32(_BN) * log_k)
            r[t] = r[t] - q

    for t in range(2):
        out_ref[:, t * _D:(t + 1) * _D] = z[t] - r[t]
    kl_ref[...] = (kl[0] + kl[1]).reshape(1, 1, 1)
    com_ref[...] = (jnp.sum(r[0] * r[0])
                    + jnp.sum(r[1] * r[1])).reshape(1, 1, 1)


def kernel(top_latent, bottom_latent, top_codebook, bottom_codebook):
    nb = _ROWS // _BN

    quant, kl_parts, com_parts = pl.pallas_call(
        _hqvae_block,
        grid=(nb,),
        in_specs=[
            pl.BlockSpec((_BN, _D), lambda i: (i, 0)),
            pl.BlockSpec((_BN, _D), lambda i: (i, 0)),
            pl.BlockSpec((_VOCAB, _D), lambda i: (0, 0)),
            pl.BlockSpec((_VOCAB, _D), lambda i: (0, 0)),
        ],
        out_specs=(
            pl.BlockSpec((_BN, 2 * _D), lambda i: (i, 0)),
            pl.BlockSpec((1, 1, 1), lambda i: (i, 0, 0)),
            pl.BlockSpec((1, 1, 1), lambda i: (i, 0, 0)),
        ),
        out_shape=(
            jax.ShapeDtypeStruct((_ROWS, 2 * _D), jnp.float32),
            jax.ShapeDtypeStruct((nb, 1, 1), jnp.float32),
            jax.ShapeDtypeStruct((nb, 1, 1), jnp.float32),
        ),
        scratch_shapes=[
            pltpu.VMEM((2, _VOCAB, _D), jnp.bfloat16),
            pltpu.VMEM((2, _VOCAB), jnp.float32),
            pltpu.VMEM((2, _VOCAB, _D), jnp.bfloat16),
            pltpu.VMEM((2, _VOCAB), jnp.float32),
        ],
    )(top_latent.reshape(_ROWS, _D), bottom_latent.reshape(_ROWS, _D),
      top_codebook, bottom_codebook)

    loss = (jnp.sum(com_parts) / np.float32(_ROWS * _D)
            + np.float32(_KL_WEIGHT) * jnp.sum(kl_parts) / np.float32(_ROWS))
    return (loss, quant.reshape(8, 1024, 2 * _D))


# speedup vs baseline: 1.0862x; 1.0312x over previous
import numpy as np
import jax
import jax.numpy as jnp
from jax import lax
from jax.experimental import pallas as pl
from jax.experimental.pallas import tpu as pltpu

_VOCAB = 1024
_D = 256
_LEVELS = 2
_KL_WEIGHT = 0.001
_ROWS = 8 * 1024
_BN = 1024
_LOG2E = np.float32(np.log2(np.e))


def _hqvae_block(zt_ref, zb_ref, ct_ref, cbm_ref, out_ref, kl_ref, com_ref,
                 cb2_ref, c2row_ref, caug_ref, c2n_ref):
    @pl.when(pl.program_id(0) == 0)
    def _prep():
        for t, cref in enumerate((ct_ref, cbm_ref)):
            cb = cref[...]
            cbsq = cb * cb
            cb2_ref[t] = (cb * (2.0 * _LOG2E)).astype(jnp.bfloat16)
            c2row_ref[t:t + 1, :] = lax.dot_general(
                jnp.full((1, _D), _LOG2E, jnp.float32), cbsq,
                (((1,), (1,)), ((), ())),
                preferred_element_type=jnp.float32)
            caug_ref[t] = cb.astype(jnp.bfloat16)
            c2n_ref[t:t + 1, :] = lax.dot_general(
                jnp.ones((1, _D), jnp.float32), cbsq,
                (((1,), (1,)), ((), ())),
                preferred_element_type=jnp.float32).astype(jnp.bfloat16)

    log_k = np.float32(np.log(float(_VOCAB)))

    z = [zt_ref[...], zb_ref[...]]
    r = [z[0], z[1]]
    kl = [jnp.float32(0.0), jnp.float32(0.0)]
    for _ in range(_LEVELS):
        e = [None, None]
        u = [None, None]
        for t in range(2):
            e[t] = jnp.exp2(lax.dot_general(
                r[t].astype(jnp.bfloat16), cb2_ref[t],
                (((1,), (1,)), ((), ())),
                preferred_element_type=jnp.float32)
                - c2row_ref[t:t + 1, :]).astype(jnp.bfloat16)
        for t in range(2):
            u[t] = lax.dot_general(
                e[t], caug_ref[t], (((1,), (0,)), ((), ())),
                preferred_element_type=jnp.float32)
        for t in range(2):
            zsum = jnp.sum(e[t], axis=1, keepdims=True,
                           dtype=jnp.float32)
            ec2 = jnp.sum(e[t] * c2n_ref[t:t + 1, :], axis=1,
                          keepdims=True, dtype=jnp.float32)
            inv = 1.0 / zsum
            q = u[t] * inv
            kl[t] = (kl[t]
                     + 2.0 * jnp.sum(r[t] * q)
                     - jnp.sum(ec2 * inv)
                     - jnp.sum(jnp.log(zsum))
                     + np.float32(_BN) * log_k)
            r[t] = r[t] - q

    for t in range(2):
        out_ref[:, t * _D:(t + 1) * _D] = z[t] - r[t]
    kl_ref[...] = (kl[0] + kl[1]).reshape(1, 1, 1)
    com_ref[...] = (jnp.sum(r[0] * r[0])
                    + jnp.sum(r[1] * r[1])).reshape(1, 1, 1)


def kernel(top_latent, bottom_latent, top_codebook, bottom_codebook):
    nb = _ROWS // _BN

    quant, kl_parts, com_parts = pl.pallas_call(
        _hqvae_block,
        grid=(nb,),
        in_specs=[
            pl.BlockSpec((_BN, _D), lambda i: (i, 0)),
            pl.BlockSpec((_BN, _D), lambda i: (i, 0)),
            pl.BlockSpec((_VOCAB, _D), lambda i: (0, 0)),
            pl.BlockSpec((_VOCAB, _D), lambda i: (0, 0)),
        ],
        out_specs=(
            pl.BlockSpec((_BN, 2 * _D), lambda i: (i, 0)),
            pl.BlockSpec((1, 1, 1), lambda i: (i, 0, 0)),
            pl.BlockSpec((1, 1, 1), lambda i: (i, 0, 0)),
        ),
        out_shape=(
            jax.ShapeDtypeStruct((_ROWS, 2 * _D), jnp.float32),
            jax.ShapeDtypeStruct((nb, 1, 1), jnp.float32),
            jax.ShapeDtypeStruct((nb, 1, 1), jnp.float32),
        ),
        scratch_shapes=[
            pltpu.VMEM((2, _VOCAB, _D), jnp.bfloat16),
            pltpu.VMEM((2, _VOCAB), jnp.float32),
            pltpu.VMEM((2, _VOCAB, _D), jnp.bfloat16),
            pltpu.VMEM((2, _VOCAB), jnp.bfloat16),
        ],
    )(top_latent.reshape(_ROWS, _D), bottom_latent.reshape(_ROWS, _D),
      top_codebook, bottom_codebook)

    loss = (jnp.sum(com_parts) / np.float32(_ROWS * _D)
            + np.float32(_KL_WEIGHT) * jnp.sum(kl_parts) / np.float32(_ROWS))
    return (loss, quant.reshape(8, 1024, 2 * _D))
